# baseline (speedup 1.0000x reference)
import functools

import jax
import jax.numpy as jnp
from jax import lax
from jax.experimental import pallas as pl
from jax.experimental.pallas import tpu as pltpu
from jax.experimental.pallas import tpu_sc as plsc

_N = 8192
_K = 8192
_D = 32
_T = 1024
_GRID = _N // _T
_CHUNK = 2048
_DP = 128
_CH = 128


def _vq_tc_body(zsq_ref, cbsq_ref, z_ref, cb_ref,
                idx_ref, loss_ref, perp_ref, cbp_ref, hist_ref, dsum_ref):
    i = pl.program_id(0)

    @pl.when(i == 0)
    def _init():
        hist_ref[...] = jnp.zeros_like(hist_ref)
        dsum_ref[0, 0] = 0.0
        cbp_ref[...] = jnp.concatenate(
            [cb_ref[...], jnp.zeros((_K, _DP - _D), jnp.float32)], axis=1)

    z = z_ref[...]
    cb = cb_ref[...]
    scores2 = jax.lax.dot_general(
        z + z, cb, (((1,), (1,)), ((), ())),
        preferred_element_type=jnp.float32)
    dist = (zsq_ref[...] + cbsq_ref[...]) - scores2
    col = jax.lax.broadcasted_iota(jnp.int32, (_T, _K), 1)
    acc_bf = acc_f = acc_i = None
    for c in range(0, _K, _CHUNK):
        dc = dist[:, c:c + _CHUNK]
        cc = col[:, c:c + _CHUNK]
        dmin = jnp.min(dc, axis=1, keepdims=True)
        imin = jnp.min(jnp.where(dc == dmin, cc, _K), axis=1)
        dmin = dmin[:, 0]
        if acc_bf is None:
            acc_f, acc_i = dmin, imin
            acc_bf = dmin.astype(jnp.bfloat16).astype(jnp.float32)
        else:
            keep = (acc_bf < dmin) | ((acc_bf == dmin) & (acc_i < imin))
            acc_f = jnp.where(keep, acc_f, dmin)
            acc_i = jnp.where(keep, acc_i, imin)
            acc_bf = jnp.where(keep, acc_bf, dmin)
            acc_bf = acc_bf.astype(jnp.bfloat16).astype(jnp.float32)
    idx = acc_i
    idx_ref[...] = idx
    onehot = jnp.where(col == idx[:, None], 1.0, 0.0)
    hist_ref[...] += jax.lax.dot_general(
        jnp.ones((1, _T), jnp.float32), onehot,
        (((1,), (0,)), ((), ())), preferred_element_type=jnp.float32)
    dsum_ref[0, 0] += jnp.sum(acc_f)

    @pl.when(i == _GRID - 1)
    def _finish():
        loss_ref[0, 0] = 1.25 * dsum_ref[0, 0] / (_N * _D)
        p = hist_ref[...] * (1.0 / _N)
        s = jnp.sum(p * jnp.log(p + 1e-10))
        perp_ref[0, 0] = jnp.exp(-s)


_vq_tc = pl.pallas_call(
    _vq_tc_body,
    grid=(_GRID,),
    in_specs=[
        pl.BlockSpec((_T, 1), lambda i: (i, 0)),
        pl.BlockSpec((1, _K), lambda i: (0, 0)),
        pl.BlockSpec((_T, _D), lambda i: (i, 0)),
        pl.BlockSpec((_K, _D), lambda i: (0, 0)),
    ],
    out_specs=[
        pl.BlockSpec((_T,), lambda i: (i,)),
        pl.BlockSpec((1, 1), lambda i: (0, 0), memory_space=pltpu.SMEM),
        pl.BlockSpec((1, 1), lambda i: (0, 0), memory_space=pltpu.SMEM),
        pl.BlockSpec((_K, _DP), lambda i: (0, 0)),
    ],
    out_shape=[
        jax.ShapeDtypeStruct((_N,), jnp.int32),
        jax.ShapeDtypeStruct((1, 1), jnp.float32),
        jax.ShapeDtypeStruct((1, 1), jnp.float32),
        jax.ShapeDtypeStruct((_K, _DP), jnp.float32),
    ],
    scratch_shapes=[
        pltpu.VMEM((1, _K), jnp.float32),
        pltpu.SMEM((1, 1), jnp.float32),
    ],
    compiler_params=pltpu.CompilerParams(dimension_semantics=("arbitrary",)),
)


@functools.cache
def _make_sc_gather():
    nc, ns = 2, 16
    nw = nc * ns
    b_per_w = _N // nw
    n_ch = b_per_w // _CH
    mesh = plsc.VectorSubcoreMesh(
        core_axis_name="c", subcore_axis_name="s",
        num_cores=nc, num_subcores=ns)

    @functools.partial(
        pl.kernel, mesh=mesh,
        out_type=jax.ShapeDtypeStruct((_N, _DP), jnp.float32),
        scratch_types=[
            pltpu.VMEM((n_ch, _CH), jnp.int32),
            pltpu.VMEM((b_per_w, _DP), jnp.float32),
            pltpu.SemaphoreType.DMA,
        ],
    )
    def _gather(cb_hbm, idx_hbm, out_hbm, idx_v, rows_v, sem):
        wid = lax.axis_index("s") * nc + lax.axis_index("c")
        base = wid * b_per_w
        pltpu.sync_copy(
            idx_hbm.at[pl.ds(wid * n_ch, n_ch)], idx_v)
        for j in range(n_ch):
            pltpu.async_copy(
                cb_hbm.at[idx_v.at[j]],
                rows_v.at[pl.ds(j * _CH, _CH)], sem).wait()
        pltpu.sync_copy(rows_v, out_hbm.at[pl.ds(base, b_per_w)])

    return _gather


def kernel(z_e, codebook):
    zsq = jnp.sum(z_e ** 2, axis=1, keepdims=True)
    cbsq = jnp.sum(codebook ** 2, axis=1).reshape(1, _K)
    idx, loss, perp, cb_pad = _vq_tc(zsq, cbsq, z_e, codebook)
    q_pad = _make_sc_gather()(cb_pad, idx.reshape(_N // _CH, _CH))
    return (loss[0, 0], q_pad[:, :_D], perp[0, 0])


# speedup vs baseline: 1.0455x; 1.0455x over previous
import functools

import jax
import jax.numpy as jnp
from jax import lax
from jax.experimental import pallas as pl
from jax.experimental.pallas import tpu as pltpu
from jax.experimental.pallas import tpu_sc as plsc

_N = 8192
_K = 8192
_D = 32
_T = 2048
_GRID = _N // _T
_CHUNK = 2048
_DP = 128
_CH = 128


def _vq_tc_body(zsq_ref, cbsq_ref, z_ref, cb_ref,
                idx_ref, loss_ref, perp_ref, cbp_ref, hist_ref, dsum_ref):
    i = pl.program_id(0)

    @pl.when(i == 0)
    def _init():
        hist_ref[...] = jnp.zeros_like(hist_ref)
        dsum_ref[0, 0] = 0.0
        cbp_ref[...] = jnp.concatenate(
            [cb_ref[...], jnp.zeros((_K, _DP - _D), jnp.float32)], axis=1)

    z = z_ref[...]
    cb = cb_ref[...]
    scores = jax.lax.dot_general(
        z, cb, (((1,), (1,)), ((), ())),
        preferred_element_type=jnp.float32)
    dist = (zsq_ref[...] + cbsq_ref[...]) - 2.0 * scores
    col = jax.lax.broadcasted_iota(jnp.int32, (_T, _K), 1)
    acc_bf = acc_f = acc_i = None
    for c in range(0, _K, _CHUNK):
        dc = dist[:, c:c + _CHUNK]
        cc = col[:, c:c + _CHUNK]
        dmin = jnp.min(dc, axis=1, keepdims=True)
        imin = jnp.min(jnp.where(dc == dmin, cc, _K), axis=1)
        dmin = dmin[:, 0]
        if acc_bf is None:
            acc_f, acc_i = dmin, imin
            acc_bf = dmin.astype(jnp.bfloat16).astype(jnp.float32)
        else:
            keep = (acc_bf < dmin) | ((acc_bf == dmin) & (acc_i < imin))
            acc_f = jnp.where(keep, acc_f, dmin)
            acc_i = jnp.where(keep, acc_i, imin)
            acc_bf = jnp.where(keep, acc_bf, dmin)
            acc_bf = acc_bf.astype(jnp.bfloat16).astype(jnp.float32)
    idx = acc_i
    idx_ref[...] = idx
    onehot = jnp.where(col == idx[:, None], 1.0, 0.0)
    hist_ref[...] += jax.lax.dot_general(
        jnp.ones((1, _T), jnp.float32), onehot,
        (((1,), (0,)), ((), ())), preferred_element_type=jnp.float32)
    dsum_ref[0, 0] += jnp.sum(acc_f)

    @pl.when(i == _GRID - 1)
    def _finish():
        loss_ref[0, 0] = 1.25 * dsum_ref[0, 0] / (_N * _D)
        p = hist_ref[...] * (1.0 / _N)
        s = jnp.sum(p * jnp.log(p + 1e-10))
        perp_ref[0, 0] = jnp.exp(-s)


_vq_tc = pl.pallas_call(
    _vq_tc_body,
    grid=(_GRID,),
    in_specs=[
        pl.BlockSpec((_T, 1), lambda i: (i, 0)),
        pl.BlockSpec((1, _K), lambda i: (0, 0)),
        pl.BlockSpec((_T, _D), lambda i: (i, 0)),
        pl.BlockSpec((_K, _D), lambda i: (0, 0)),
    ],
    out_specs=[
        pl.BlockSpec((_T,), lambda i: (i,)),
        pl.BlockSpec((1, 1), lambda i: (0, 0), memory_space=pltpu.SMEM),
        pl.BlockSpec((1, 1), lambda i: (0, 0), memory_space=pltpu.SMEM),
        pl.BlockSpec((_K, _DP), lambda i: (0, 0)),
    ],
    out_shape=[
        jax.ShapeDtypeStruct((_N,), jnp.int32),
        jax.ShapeDtypeStruct((1, 1), jnp.float32),
        jax.ShapeDtypeStruct((1, 1), jnp.float32),
        jax.ShapeDtypeStruct((_K, _DP), jnp.float32),
    ],
    scratch_shapes=[
        pltpu.VMEM((1, _K), jnp.float32),
        pltpu.SMEM((1, 1), jnp.float32),
    ],
    compiler_params=pltpu.CompilerParams(dimension_semantics=("arbitrary",)),
)


@functools.cache
def _make_sc_gather():
    nc, ns = 2, 16
    nw = nc * ns
    b_per_w = _N // nw
    n_ch = b_per_w // _CH
    mesh = plsc.VectorSubcoreMesh(
        core_axis_name="c", subcore_axis_name="s",
        num_cores=nc, num_subcores=ns)

    @functools.partial(
        pl.kernel, mesh=mesh,
        out_type=jax.ShapeDtypeStruct((_N, _DP), jnp.float32),
        scratch_types=[
            pltpu.VMEM((n_ch, _CH), jnp.int32),
            pltpu.VMEM((b_per_w, _DP), jnp.float32),
            pltpu.SemaphoreType.DMA,
        ],
    )
    def _gather(cb_hbm, idx_hbm, out_hbm, idx_v, rows_v, sem):
        wid = lax.axis_index("s") * nc + lax.axis_index("c")
        base = wid * b_per_w
        pltpu.sync_copy(
            idx_hbm.at[pl.ds(wid * n_ch, n_ch)], idx_v)
        for j in range(n_ch):
            pltpu.async_copy(
                cb_hbm.at[idx_v.at[j]],
                rows_v.at[pl.ds(j * _CH, _CH)], sem).wait()
        pltpu.sync_copy(rows_v, out_hbm.at[pl.ds(base, b_per_w)])

    return _gather


def kernel(z_e, codebook):
    zsq = jnp.sum(z_e ** 2, axis=1, keepdims=True)
    cbsq = jnp.sum(codebook ** 2, axis=1).reshape(1, _K)
    idx, loss, perp, cb_pad = _vq_tc(zsq, cbsq, z_e, codebook)
    q_pad = _make_sc_gather()(cb_pad, idx.reshape(_N // _CH, _CH))
    return (loss[0, 0], q_pad[:, :_D], perp[0, 0])
